# baseline (speedup 1.0000x reference)
import jax
import jax.numpy as jnp
from jax.experimental import pallas as pl
from jax.experimental.pallas import tpu as pltpu

_ROWS = 824


def _copy_kernel(x_ref, o_ref):
    o_ref[...] = x_ref[...]


def kernel(x):
    b, c, f = x.shape
    x2 = x.reshape(b * c, f)
    out = pl.pallas_call(
        _copy_kernel,
        out_shape=jax.ShapeDtypeStruct(x2.shape, x2.dtype),
        grid=(pl.cdiv(b * c, _ROWS),),
        in_specs=[pl.BlockSpec((_ROWS, f), lambda i: (i, 0))],
        out_specs=pl.BlockSpec((_ROWS, f), lambda i: (i, 0)),
        compiler_params=pltpu.CompilerParams(vmem_limit_bytes=100 * 1024 * 1024),
    )(x2)
    return out.reshape(b, c, f)


# speedup vs baseline: 1.0147x; 1.0147x over previous
import jax
import jax.numpy as jnp
from jax.experimental import pallas as pl
from jax.experimental.pallas import tpu as pltpu

_ROWS = 896


def _copy_kernel(x_ref, o_ref):
    o_ref[...] = x_ref[...]


def kernel(x):
    b, c, f = x.shape
    x2 = x.reshape(b * c, f)
    out = pl.pallas_call(
        _copy_kernel,
        out_shape=jax.ShapeDtypeStruct(x2.shape, x2.dtype),
        grid=(pl.cdiv(b * c, _ROWS),),
        in_specs=[pl.BlockSpec((_ROWS, f), lambda i: (i, 0))],
        out_specs=pl.BlockSpec((_ROWS, f), lambda i: (i, 0)),
    )(x2)
    return out.reshape(b, c, f)
